# Initial kernel scaffold; baseline (speedup 1.0000x reference)
import jax
import jax.numpy as jnp
from jax.experimental import pallas as pl


def kernel(coord, nlist, davg, dstd, ew0, eb0, ew1, eb1, ew2, eb2, fw0, fb0, fw1, fb1, fw2, fb2, fwo, fbo):
    raise NotImplementedError("write your pallas kernel here")


# speedup vs baseline: 1.2023x; 1.2023x over previous
import functools
import jax
import jax.numpy as jnp
from jax import lax
from jax.experimental import pallas as pl
from jax.experimental.pallas import tpu as pltpu
from jax.experimental.pallas import tpu_sc as plsc

N = 10000
NNEI = 64
RCUT = 6.0
RCUT_SMTH = 0.5
NE = N * NNEI

B = 40
NB = N // B
E = B * NNEI

NC = 2
NS = 16
NW = NC * NS
EPW = NE // NW
SUB = 80
NSUB = 25
CH = SUB * NSUB
NCHUNK = EPW // CH
NPAD = 10240
RPT = NPAD // NS

_MESH = dict(core_axis_name="c", subcore_axis_name="s", num_cores=NC,
             num_subcores=NS)


@functools.cache
def _get_sc_gather():
  return functools.partial(
      pl.kernel,
      out_type=jax.ShapeDtypeStruct((NE, 8), jnp.float32),
      mesh=plsc.VectorSubcoreMesh(**_MESH),
      compiler_params=pltpu.CompilerParams(use_tc_tiling_on_sc=False),
      scratch_types=[
          pltpu.VMEM((NSUB, SUB), jnp.int32),
          pltpu.VMEM((CH, 8), jnp.float32),
          pltpu.SemaphoreType.DMA,
          pltpu.SemaphoreType.DMA,
      ],
  )(_sc_gather_body)


def _sc_gather_body(coordp_hbm, nlist_hbm, out_hbm, idx_v, rows_v, gsem, osem):
  c = lax.axis_index("c")
  s = lax.axis_index("s")
  wid = s * NC + c

  def chunk_body(j, carry):
    cb = wid * EPW + j * CH
    pltpu.sync_copy(nlist_hbm.at[wid, j], idx_v)
    descs = []
    for j2 in range(NSUB):
      descs.append(
          pltpu.async_copy(coordp_hbm.at[idx_v.at[j2]],
                           rows_v.at[pl.ds(j2 * SUB, SUB)], gsem))
    for d in descs:
      d.wait()
    pltpu.async_copy(rows_v, out_hbm.at[pl.ds(cb, CH)], osem).wait()
    return carry

  lax.fori_loop(0, NCHUNK, chunk_body, 0)


@functools.cache
def _get_sc_scatter():
  return functools.partial(
      pl.kernel,
      out_type=jax.ShapeDtypeStruct((NC * NPAD, 8), jnp.float32),
      mesh=plsc.VectorSubcoreMesh(**_MESH),
      compiler_params=pltpu.CompilerParams(use_tc_tiling_on_sc=False),
      scratch_types=[
          pltpu.VMEM((NSUB, SUB), jnp.int32),
          pltpu.VMEM((CH, 8), jnp.float32),
          pltpu.VMEM_SHARED((NPAD, 8), jnp.float32),
          pltpu.SemaphoreType.DMA,
      ],
  )(_sc_scatter_body)


def _sc_scatter_body(grows_hbm, nlist_hbm, zeros_hbm, out_hbm,
                     idx_v, rows_v, acc_sh, sem):
  c = lax.axis_index("c")
  s = lax.axis_index("s")
  wid = s * NC + c

  @pl.when(s == 0)
  def _():
    pltpu.sync_copy(zeros_hbm, acc_sh)

  plsc.subcore_barrier()

  def chunk_body(j, carry):
    cb = wid * EPW + j * CH
    pltpu.sync_copy(nlist_hbm.at[wid, j], idx_v)
    pltpu.async_copy(grows_hbm.at[pl.ds(cb, CH)], rows_v, sem).wait()
    for j2 in range(NSUB):
      pltpu.sync_copy(rows_v.at[pl.ds(j2 * SUB, SUB)],
                      acc_sh.at[idx_v.at[j2]], add=True)
    return carry

  lax.fori_loop(0, NCHUNK, chunk_body, 0)
  plsc.subcore_barrier()
  pltpu.sync_copy(acc_sh.at[pl.ds(s * RPT, RPT)],
                  out_hbm.at[pl.ds(c * NPAD + s * RPT, RPT)])


def _dense_kernel(rows_ref, coordb_ref, davg_ref, dstd_ref,
                  ew0_ref, eb0_ref, ew1_ref, eb1_ref, ew2_ref, eb2_ref,
                  ew1t_ref, ew2t_ref,
                  fw0p_ref, fb0_ref, fw0pt_ref,
                  fw1_ref, fb1_ref, fw1t_ref,
                  fw2_ref, fb2_ref, fw2t_ref,
                  fwot_ref, fbo_ref,
                  ae_ref, esum_ref, selfsum_ref, grows_ref):
  f32 = jnp.float32
  r8 = rows_ref[...]
  xg = r8[:, 0:1].reshape(B, NNEI)
  yg = r8[:, 1:2].reshape(B, NNEI)
  zg = r8[:, 2:3].reshape(B, NNEI)
  cb = coordb_ref[...]
  rx = xg - cb[:, 0:1]
  ry = yg - cb[:, 1:2]
  rz = zg - cb[:, 2:3]
  rr2 = rx * rx + ry * ry + rz * rz
  rr = jnp.sqrt(rr2 + 1e-12)
  u_raw = (rr - RCUT_SMTH) / (RCUT - RCUT_SMTH)
  uu = jnp.clip(u_raw, 0.0, 1.0)
  uu2 = uu * uu
  sw = (uu2 * uu) * ((-6.0 * uu + 15.0) * uu - 10.0) + 1.0
  inv_r = 1.0 / jnp.maximum(rr, 1e-2)
  sr = sw * inv_r
  sx = sr * inv_r
  dv = davg_ref[...]
  sd = dstd_ref[...]
  R0 = (sr - dv[:, 0:1]) / sd[:, 0:1]
  R1 = (sx * rx - dv[:, 1:2]) / sd[:, 1:2]
  R2 = (sx * ry - dv[:, 2:3]) / sd[:, 2:3]
  R3 = (sx * rz - dv[:, 3:4]) / sd[:, 3:4]

  Rn = jnp.concatenate(
      [R0.reshape(E, 1), R1.reshape(E, 1), R2.reshape(E, 1),
       R3.reshape(E, 1)], axis=1)

  xx = Rn[:, 0:1]
  h0 = jnp.tanh(xx * ew0_ref[...] + eb0_ref[...])
  z1t = jnp.tanh(
      jnp.dot(h0, ew1_ref[...], preferred_element_type=f32) + eb1_ref[...])
  h1 = jnp.concatenate([h0, h0], axis=1) + z1t
  z2t = jnp.tanh(
      jnp.dot(h1, ew2_ref[...], preferred_element_type=f32) + eb2_ref[...])
  G = jnp.concatenate([h1, h1], axis=1) + z2t

  inv_n = f32(1.0 / NNEI)
  gra = []
  for a in range(4):
    prod = G * Rn[:, a:a + 1]
    gra.append(jnp.sum(prod.reshape(B, NNEI, 32), axis=1) * inv_n)
  dks = []
  for k in range(4):
    acc = gra[0] * gra[0][:, k:k + 1]
    for a in range(1, 4):
      acc = acc + gra[a] * gra[a][:, k:k + 1]
    dks.append(acc)
  dd = jnp.concatenate(dks, axis=1)

  f0 = jnp.tanh(
      jnp.dot(dd, fw0p_ref[...], preferred_element_type=f32) + fb0_ref[...])
  t1 = jnp.tanh(
      jnp.dot(f0, fw1_ref[...], preferred_element_type=f32) + fb1_ref[...])
  f1 = f0 + t1
  t2 = jnp.tanh(
      jnp.dot(f1, fw2_ref[...], preferred_element_type=f32) + fb2_ref[...])
  f2 = f1 + t2
  aev = jnp.sum(f2 * fwot_ref[...], axis=1, keepdims=True) + fbo_ref[...]
  ae_ref[...] = aev

  @pl.when(pl.program_id(0) == 0)
  def _():
    esum_ref[...] = jnp.zeros_like(esum_ref)

  esum_ref[...] += jnp.sum(aev).reshape(1, 1)

  g_f2 = jnp.broadcast_to(fwot_ref[...], (B, 64))
  g_f1 = g_f2 + jnp.dot((1.0 - t2 * t2) * g_f2, fw2t_ref[...],
                        preferred_element_type=f32)
  g_f0 = g_f1 + jnp.dot((1.0 - t1 * t1) * g_f1, fw1t_ref[...],
                        preferred_element_type=f32)
  g_dd = jnp.dot((1.0 - f0 * f0) * g_f0, fw0pt_ref[...],
                 preferred_element_type=f32)

  iota32 = lax.broadcasted_iota(jnp.int32, (1, 32), 1)
  g_gra = []
  for a in range(4):
    acc = None
    for k in range(4):
      g_dk = g_dd[:, 32 * k:32 * k + 32]
      term = g_dk * gra[a][:, k:k + 1]
      s_ak = jnp.sum(g_dk * gra[a], axis=1, keepdims=True)
      term = term + jnp.where(iota32 == k, s_ak, f32(0.0))
      acc = term if acc is None else acc + term
    g_gra.append(acc)

  g_G = None
  g_Rn_cols = []
  for a in range(4):
    bg = jnp.broadcast_to(g_gra[a][:, None, :], (B, NNEI, 32)).reshape(E, 32)
    contrib = Rn[:, a:a + 1] * bg
    g_G = contrib if g_G is None else g_G + contrib
    g_Rn_cols.append(jnp.sum(G * bg, axis=1, keepdims=True) * inv_n)
  g_G = g_G * inv_n

  q2 = (1.0 - z2t * z2t) * g_G
  g_h1 = g_G[:, :16] + g_G[:, 16:] + jnp.dot(
      q2, ew2t_ref[...], preferred_element_type=f32)
  q1 = (1.0 - z1t * z1t) * g_h1
  g_h0 = g_h1[:, :8] + g_h1[:, 8:] + jnp.dot(
      q1, ew1t_ref[...], preferred_element_type=f32)
  q0 = (1.0 - h0 * h0) * g_h0
  g_xx = jnp.sum(q0 * ew0_ref[...], axis=1, keepdims=True)
  g_Rn_cols[0] = g_Rn_cols[0] + g_xx

  g0 = g_Rn_cols[0].reshape(B, NNEI) / sd[:, 0:1]
  g1 = g_Rn_cols[1].reshape(B, NNEI) / sd[:, 1:2]
  g2 = g_Rn_cols[2].reshape(B, NNEI) / sd[:, 2:3]
  g3 = g_Rn_cols[3].reshape(B, NNEI) / sd[:, 3:4]

  g_sr = g0 + inv_r * (g1 * rx + g2 * ry + g3 * rz)
  g_invr = sr * (g1 * rx + g2 * ry + g3 * rz) + g_sr * sw
  g_sw = g_sr * inv_r
  dinvr = jnp.where(rr > 1e-2, -inv_r * inv_r, f32(0.0))
  um1 = uu - 1.0
  dsw_duu = -30.0 * uu2 * um1 * um1
  duu = jnp.where((u_raw > 0.0) & (u_raw < 1.0), f32(1.0 / (RCUT - RCUT_SMTH)),
                  f32(0.0))
  g_rr = g_sw * dsw_duu * duu + g_invr * dinvr
  grr_over = g_rr / rr
  gxp = g1 * sx + rx * grr_over
  gyp = g2 * sx + ry * grr_over
  gzp = g3 * sx + rz * grr_over

  selfsum_ref[...] = jnp.concatenate(
      [jnp.sum(gxp, axis=1, keepdims=True),
       jnp.sum(gyp, axis=1, keepdims=True),
       jnp.sum(gzp, axis=1, keepdims=True),
       jnp.zeros((B, 1), f32)], axis=1)
  grows_ref[...] = jnp.concatenate(
      [gxp.reshape(E, 1), gyp.reshape(E, 1), gzp.reshape(E, 1),
       jnp.zeros((E, 5), f32)], axis=1)


def _full(shape):
  return pl.BlockSpec(shape, lambda i: tuple(0 for _ in shape))


DENSE_GRID = (NB,)
DENSE_IN_SPECS = [
    pl.BlockSpec((E, 8), lambda i: (i, 0)),
    pl.BlockSpec((B, 8), lambda i: (i, 0)),
    _full((1, 4)), _full((1, 4)),
    _full((1, 8)), _full((1, 8)),
    _full((8, 16)), _full((1, 16)),
    _full((16, 32)), _full((1, 32)),
    _full((16, 8)), _full((32, 16)),
    _full((128, 64)), _full((1, 64)), _full((64, 128)),
    _full((64, 64)), _full((1, 64)), _full((64, 64)),
    _full((64, 64)), _full((1, 64)), _full((64, 64)),
    _full((1, 64)), _full((1, 1)),
]
DENSE_OUT_SPECS = [
    pl.BlockSpec((B, 1), lambda i: (i, 0)),
    pl.BlockSpec((1, 1), lambda i: (0, 0)),
    pl.BlockSpec((B, 4), lambda i: (i, 0)),
    pl.BlockSpec((E, 8), lambda i: (i, 0)),
]
DENSE_OUT_SHAPES = [
    jax.ShapeDtypeStruct((N, 1), jnp.float32),
    jax.ShapeDtypeStruct((1, 1), jnp.float32),
    jax.ShapeDtypeStruct((N, 4), jnp.float32),
    jax.ShapeDtypeStruct((NE, 8), jnp.float32),
]

_dense_call = pl.pallas_call(
    _dense_kernel,
    grid=DENSE_GRID,
    in_specs=DENSE_IN_SPECS,
    out_specs=DENSE_OUT_SPECS,
    out_shape=DENSE_OUT_SHAPES,
)


def _combine_kernel(selfsum_ref, parts_ref, force_ref):
  p = parts_ref[...]
  force_ref[...] = (selfsum_ref[...] - p[:N, :4] - p[NPAD:NPAD + N, :4])


_combine_call = pl.pallas_call(
    _combine_kernel,
    out_shape=jax.ShapeDtypeStruct((N, 4), jnp.float32),
)


@jax.jit
def kernel(coord, nlist, davg, dstd, ew0, eb0, ew1, eb1, ew2, eb2,
           fw0, fb0, fw1, fb1, fw2, fb2, fwo, fbo):
  f32 = jnp.float32
  coordp = jnp.zeros((N, 8), f32).at[:, :3].set(coord.astype(f32))
  nlist_i = nlist.astype(jnp.int32).reshape(NW, NCHUNK, NSUB, SUB)

  rows = _get_sc_gather()(coordp, nlist_i)

  fw0p = fw0.reshape(32, 4, 64).transpose(1, 0, 2).reshape(128, 64)
  args = (
      rows, coordp,
      davg.reshape(1, 4), dstd.reshape(1, 4),
      ew0, eb0.reshape(1, 8),
      ew1, eb1.reshape(1, 16),
      ew2, eb2.reshape(1, 32),
      ew1.T, ew2.T,
      fw0p, fb0.reshape(1, 64), fw0p.T,
      fw1, fb1.reshape(1, 64), fw1.T,
      fw2, fb2.reshape(1, 64), fw2.T,
      fwo.reshape(1, 64), fbo.reshape(1, 1),
  )
  ae2, esum, selfsum, grows = _dense_call(*args)

  parts = _get_sc_scatter()(grows, nlist_i, jnp.zeros((NPAD, 8), f32))
  force4 = _combine_call(selfsum, parts)

  return (esum[0, 0], ae2.reshape(N), force4[:, :3])


# speedup vs baseline: 1.6520x; 1.3741x over previous
import functools
import jax
import jax.numpy as jnp
from jax import lax
from jax.experimental import pallas as pl
from jax.experimental.pallas import tpu as pltpu
from jax.experimental.pallas import tpu_sc as plsc

N = 10000
NNEI = 64
RCUT = 6.0
RCUT_SMTH = 0.5
NE = N * NNEI

B = 40
NB = N // B
E = B * NNEI

NC = 2
NS = 16
NW = NC * NS
EPW = NE // NW
SUB = 80
NSUB = 25
CH = SUB * NSUB
NCHUNK = EPW // CH
NPAD = 10240
RPT = NPAD // NS

_MESH = dict(core_axis_name="c", subcore_axis_name="s", num_cores=NC,
             num_subcores=NS)


@functools.cache
def _get_sc_gather():
  return functools.partial(
      pl.kernel,
      out_type=jax.ShapeDtypeStruct((NE, 8), jnp.float32),
      mesh=plsc.VectorSubcoreMesh(**_MESH),
      compiler_params=pltpu.CompilerParams(use_tc_tiling_on_sc=False),
      scratch_types=[
          pltpu.VMEM((NSUB, SUB), jnp.int32),
          pltpu.VMEM((CH, 8), jnp.float32),
          pltpu.SemaphoreType.DMA,
          pltpu.SemaphoreType.DMA,
      ],
  )(_sc_gather_body)


def _sc_gather_body(coordp_hbm, nlist_hbm, out_hbm, idx_v, rows_v, gsem, osem):
  c = lax.axis_index("c")
  s = lax.axis_index("s")
  wid = s * NC + c

  def chunk_body(j, carry):
    cb = wid * EPW + j * CH
    pltpu.sync_copy(nlist_hbm.at[wid, j], idx_v)
    descs = []
    for j2 in range(NSUB):
      descs.append(
          pltpu.async_copy(coordp_hbm.at[idx_v.at[j2]],
                           rows_v.at[pl.ds(j2 * SUB, SUB)], gsem))
    for d in descs:
      d.wait()
    pltpu.async_copy(rows_v, out_hbm.at[pl.ds(cb, CH)], osem).wait()
    return carry

  lax.fori_loop(0, NCHUNK, chunk_body, 0)


@functools.cache
def _get_sc_scatter():
  return functools.partial(
      pl.kernel,
      out_type=jax.ShapeDtypeStruct((NC * NPAD, 8), jnp.float32),
      mesh=plsc.VectorSubcoreMesh(**_MESH),
      compiler_params=pltpu.CompilerParams(use_tc_tiling_on_sc=False),
      scratch_types=[
          pltpu.VMEM((NSUB, SUB), jnp.int32),
          pltpu.VMEM((CH, 8), jnp.float32),
          pltpu.VMEM_SHARED((NPAD, 8), jnp.float32),
          pltpu.SemaphoreType.DMA,
      ],
  )(_sc_scatter_body)


def _sc_scatter_body(grows_hbm, nlist_hbm, zeros_hbm, out_hbm,
                     idx_v, rows_v, acc_sh, sem):
  c = lax.axis_index("c")
  s = lax.axis_index("s")
  wid = s * NC + c

  @pl.when(s == 0)
  def _():
    pltpu.sync_copy(zeros_hbm, acc_sh)

  plsc.subcore_barrier()

  def chunk_body(j, carry):
    cb = wid * EPW + j * CH
    pltpu.sync_copy(nlist_hbm.at[wid, j], idx_v)
    pltpu.async_copy(grows_hbm.at[pl.ds(cb, CH)], rows_v, sem).wait()
    for j2 in range(NSUB):
      pltpu.sync_copy(rows_v.at[pl.ds(j2 * SUB, SUB)],
                      acc_sh.at[idx_v.at[j2]], add=True)
    return carry

  lax.fori_loop(0, NCHUNK, chunk_body, 0)
  plsc.subcore_barrier()
  pltpu.sync_copy(acc_sh.at[pl.ds(s * RPT, RPT)],
                  out_hbm.at[pl.ds(c * NPAD + s * RPT, RPT)])


def _dense_kernel(rows_ref, coordb_ref, davg_ref, dstd_ref,
                  ew0_ref, ew0c_ref, eb0_ref, ew1_ref, eb1_ref, ew2_ref,
                  eb2_ref, ew1t_ref, ew2t_ref,
                  fw0p_ref, fb0_ref, fw0pt_ref,
                  fw1_ref, fb1_ref, fw1t_ref,
                  fw2_ref, fb2_ref, fw2t_ref,
                  fwot_ref, fbo_ref,
                  ae_ref, esum_ref, selfsum_ref, grows_ref):
  f32 = jnp.float32

  def mm(a, b):
    return jnp.dot(a, b, preferred_element_type=f32)

  lane_b = lax.broadcasted_iota(jnp.int32, (1, B), 1)
  lane_n = lax.broadcasted_iota(jnp.int32, (1, NNEI), 1)
  lane_e = lax.broadcasted_iota(jnp.int32, (1, E), 1)
  sub_e = lax.broadcasted_iota(jnp.int32, (E, 1), 0)
  sub_b = lax.broadcasted_iota(jnp.int32, (B, 1), 0)
  atom_of_e = sub_e // NNEI
  k_of_e = sub_e - atom_of_e * NNEI
  one = f32(1.0)
  zero = f32(0.0)
  A = jnp.where(atom_of_e == lane_b, one, zero)
  M = jnp.where(k_of_e == lane_n, one, zero)
  AT = jnp.where(sub_b == lane_e // NNEI, one, zero)
  ones_n32 = jnp.ones((NNEI, 32), f32)
  ones_32n = jnp.ones((32, NNEI), f32)

  def plane_to_q(p):
    return mm(A, p) * M

  def rows_to_plane(ec):
    return mm(AT, ec)

  r8 = rows_ref[...]
  xg = rows_to_plane(jnp.broadcast_to(r8[:, 0:1], (E, NNEI)) * M)
  yg = rows_to_plane(jnp.broadcast_to(r8[:, 1:2], (E, NNEI)) * M)
  zg = rows_to_plane(jnp.broadcast_to(r8[:, 2:3], (E, NNEI)) * M)
  cb = coordb_ref[...]
  rx = xg - cb[:, 0:1]
  ry = yg - cb[:, 1:2]
  rz = zg - cb[:, 2:3]
  rr2 = rx * rx + ry * ry + rz * rz
  rr = jnp.sqrt(rr2 + 1e-12)
  u_raw = (rr - RCUT_SMTH) / (RCUT - RCUT_SMTH)
  uu = jnp.clip(u_raw, 0.0, 1.0)
  uu2 = uu * uu
  sw = (uu2 * uu) * ((-6.0 * uu + 15.0) * uu - 10.0) + 1.0
  inv_r = 1.0 / jnp.maximum(rr, 1e-2)
  sr = sw * inv_r
  sx = sr * inv_r
  dv = davg_ref[...]
  sd = dstd_ref[...]
  R0 = (sr - dv[:, 0:1]) / sd[:, 0:1]
  R1 = (sx * rx - dv[:, 1:2]) / sd[:, 1:2]
  R2 = (sx * ry - dv[:, 2:3]) / sd[:, 2:3]
  R3 = (sx * rz - dv[:, 3:4]) / sd[:, 3:4]

  Q = [plane_to_q(R0), plane_to_q(R1), plane_to_q(R2), plane_to_q(R3)]

  w0e = jnp.broadcast_to(ew0_ref[...], (NNEI, 8))
  h0 = jnp.tanh(mm(Q[0], w0e) + eb0_ref[...])
  z1t = jnp.tanh(mm(h0, ew1_ref[...]) + eb1_ref[...])
  h1 = jnp.concatenate([h0, h0], axis=1) + z1t
  z2t = jnp.tanh(mm(h1, ew2_ref[...]) + eb2_ref[...])
  G = jnp.concatenate([h1, h1], axis=1) + z2t

  inv_n = f32(1.0 / NNEI)
  rnb = [mm(q, ones_n32) for q in Q]
  gra = [mm(AT, G * rnb[a]) * inv_n for a in range(4)]
  dks = []
  for k in range(4):
    acc = gra[0] * gra[0][:, k:k + 1]
    for a in range(1, 4):
      acc = acc + gra[a] * gra[a][:, k:k + 1]
    dks.append(acc)
  dd = jnp.concatenate(dks, axis=1)

  f0 = jnp.tanh(
      jnp.dot(dd, fw0p_ref[...], preferred_element_type=f32) + fb0_ref[...])
  t1 = jnp.tanh(
      jnp.dot(f0, fw1_ref[...], preferred_element_type=f32) + fb1_ref[...])
  f1 = f0 + t1
  t2 = jnp.tanh(
      jnp.dot(f1, fw2_ref[...], preferred_element_type=f32) + fb2_ref[...])
  f2 = f1 + t2
  aev = jnp.sum(f2 * fwot_ref[...], axis=1, keepdims=True) + fbo_ref[...]
  ae_ref[...] = aev

  @pl.when(pl.program_id(0) == 0)
  def _():
    esum_ref[...] = jnp.zeros_like(esum_ref)

  esum_ref[...] += jnp.sum(aev).reshape(1, 1)

  g_f2 = jnp.broadcast_to(fwot_ref[...], (B, 64))
  g_f1 = g_f2 + jnp.dot((1.0 - t2 * t2) * g_f2, fw2t_ref[...],
                        preferred_element_type=f32)
  g_f0 = g_f1 + jnp.dot((1.0 - t1 * t1) * g_f1, fw1t_ref[...],
                        preferred_element_type=f32)
  g_dd = jnp.dot((1.0 - f0 * f0) * g_f0, fw0pt_ref[...],
                 preferred_element_type=f32)

  iota32 = lax.broadcasted_iota(jnp.int32, (1, 32), 1)
  g_gra = []
  for a in range(4):
    acc = None
    for k in range(4):
      g_dk = g_dd[:, 32 * k:32 * k + 32]
      term = g_dk * gra[a][:, k:k + 1]
      s_ak = jnp.sum(g_dk * gra[a], axis=1, keepdims=True)
      term = term + jnp.where(iota32 == k, s_ak, f32(0.0))
      acc = term if acc is None else acc + term
    g_gra.append(acc)

  bg = [mm(A, g) for g in g_gra]
  g_G = (rnb[0] * bg[0] + rnb[1] * bg[1] + rnb[2] * bg[2]
         + rnb[3] * bg[3]) * inv_n

  q2 = (1.0 - z2t * z2t) * g_G
  g_h1 = g_G[:, :16] + g_G[:, 16:] + mm(q2, ew2t_ref[...])
  q1 = (1.0 - z1t * z1t) * g_h1
  g_h0 = g_h1[:, :8] + g_h1[:, 8:] + mm(q1, ew1t_ref[...])
  q0 = (1.0 - h0 * h0) * g_h0

  w0b = jnp.broadcast_to(ew0c_ref[...], (8, NNEI))
  sg = [mm(G * bg[a], ones_32n) for a in range(4)]
  gm0 = (sg[0] * inv_n + mm(q0, w0b)) * M
  g0 = rows_to_plane(gm0) / sd[:, 0:1]
  g1 = rows_to_plane(sg[1] * M) * inv_n / sd[:, 1:2]
  g2 = rows_to_plane(sg[2] * M) * inv_n / sd[:, 2:3]
  g3 = rows_to_plane(sg[3] * M) * inv_n / sd[:, 3:4]

  g_sr = g0 + inv_r * (g1 * rx + g2 * ry + g3 * rz)
  g_invr = sr * (g1 * rx + g2 * ry + g3 * rz) + g_sr * sw
  g_sw = g_sr * inv_r
  dinvr = jnp.where(rr > 1e-2, -inv_r * inv_r, f32(0.0))
  um1 = uu - 1.0
  dsw_duu = -30.0 * uu2 * um1 * um1
  duu = jnp.where((u_raw > 0.0) & (u_raw < 1.0), f32(1.0 / (RCUT - RCUT_SMTH)),
                  f32(0.0))
  g_rr = g_sw * dsw_duu * duu + g_invr * dinvr
  grr_over = g_rr / rr
  gxp = g1 * sx + rx * grr_over
  gyp = g2 * sx + ry * grr_over
  gzp = g3 * sx + rz * grr_over

  selfsum_ref[...] = jnp.concatenate(
      [jnp.sum(gxp, axis=1, keepdims=True),
       jnp.sum(gyp, axis=1, keepdims=True),
       jnp.sum(gzp, axis=1, keepdims=True),
       jnp.zeros((B, 1), f32)], axis=1)
  lane8 = lax.broadcasted_iota(jnp.int32, (1, 8), 1)
  wsel = [jnp.broadcast_to(jnp.where(lane8 == c, one, zero), (NNEI, 8))
          for c in range(3)]
  grows_ref[...] = (mm(plane_to_q(gxp), wsel[0])
                    + mm(plane_to_q(gyp), wsel[1])
                    + mm(plane_to_q(gzp), wsel[2]))


def _full(shape):
  return pl.BlockSpec(shape, lambda i: tuple(0 for _ in shape))


DENSE_GRID = (NB,)
DENSE_IN_SPECS = [
    pl.BlockSpec((E, 8), lambda i: (i, 0)),
    pl.BlockSpec((B, 8), lambda i: (i, 0)),
    _full((1, 4)), _full((1, 4)),
    _full((1, 8)), _full((8, 1)), _full((1, 8)),
    _full((8, 16)), _full((1, 16)),
    _full((16, 32)), _full((1, 32)),
    _full((16, 8)), _full((32, 16)),
    _full((128, 64)), _full((1, 64)), _full((64, 128)),
    _full((64, 64)), _full((1, 64)), _full((64, 64)),
    _full((64, 64)), _full((1, 64)), _full((64, 64)),
    _full((1, 64)), _full((1, 1)),
]
DENSE_OUT_SPECS = [
    pl.BlockSpec((B, 1), lambda i: (i, 0)),
    pl.BlockSpec((1, 1), lambda i: (0, 0)),
    pl.BlockSpec((B, 4), lambda i: (i, 0)),
    pl.BlockSpec((E, 8), lambda i: (i, 0)),
]
DENSE_OUT_SHAPES = [
    jax.ShapeDtypeStruct((N, 1), jnp.float32),
    jax.ShapeDtypeStruct((1, 1), jnp.float32),
    jax.ShapeDtypeStruct((N, 4), jnp.float32),
    jax.ShapeDtypeStruct((NE, 8), jnp.float32),
]

_dense_call = pl.pallas_call(
    _dense_kernel,
    grid=DENSE_GRID,
    in_specs=DENSE_IN_SPECS,
    out_specs=DENSE_OUT_SPECS,
    out_shape=DENSE_OUT_SHAPES,
)


def _combine_kernel(selfsum_ref, parts_ref, force_ref):
  p = parts_ref[...]
  force_ref[...] = (selfsum_ref[...] - p[:N, :4] - p[NPAD:NPAD + N, :4])


_combine_call = pl.pallas_call(
    _combine_kernel,
    out_shape=jax.ShapeDtypeStruct((N, 4), jnp.float32),
)


@jax.jit
def kernel(coord, nlist, davg, dstd, ew0, eb0, ew1, eb1, ew2, eb2,
           fw0, fb0, fw1, fb1, fw2, fb2, fwo, fbo):
  f32 = jnp.float32
  coordp = jnp.zeros((N, 8), f32).at[:, :3].set(coord.astype(f32))
  nlist_i = nlist.astype(jnp.int32).reshape(NW, NCHUNK, NSUB, SUB)

  rows = _get_sc_gather()(coordp, nlist_i)

  fw0p = fw0.reshape(32, 4, 64).transpose(1, 0, 2).reshape(128, 64)
  args = (
      rows, coordp,
      davg.reshape(1, 4), dstd.reshape(1, 4),
      ew0, ew0.reshape(8, 1), eb0.reshape(1, 8),
      ew1, eb1.reshape(1, 16),
      ew2, eb2.reshape(1, 32),
      ew1.T, ew2.T,
      fw0p, fb0.reshape(1, 64), fw0p.T,
      fw1, fb1.reshape(1, 64), fw1.T,
      fw2, fb2.reshape(1, 64), fw2.T,
      fwo.reshape(1, 64), fbo.reshape(1, 1),
  )
  ae2, esum, selfsum, grows = _dense_call(*args)

  parts = _get_sc_scatter()(grows, nlist_i, jnp.zeros((NPAD, 8), f32))
  force4 = _combine_call(selfsum, parts)

  return (esum[0, 0], ae2.reshape(N), force4[:, :3])
